# baseline (speedup 1.0000x reference)
import functools

import jax
import jax.numpy as jnp
from jax import lax
from jax.experimental import pallas as pl
from jax.experimental.pallas import tpu as pltpu
from jax.experimental.pallas import tpu_sc as plsc

MAX_LEN = 512
D_MODEL = 64
BATCH = 16384

_B_SC = 8192
_B_TC = BATCH - _B_SC

_NUM_CORES = 2
_NUM_SUBCORES = 16
_NUM_WORKERS = _NUM_CORES * _NUM_SUBCORES
_B_PER_W = _B_SC // _NUM_WORKERS

_CHUNKS = 2
_C = _B_PER_W // _CHUNKS

_mesh = plsc.VectorSubcoreMesh(
    core_axis_name="c", subcore_axis_name="s",
    num_cores=_NUM_CORES, num_subcores=_NUM_SUBCORES,
)


@functools.partial(
    pl.kernel,
    out_type=jax.ShapeDtypeStruct((_B_SC, 128), jnp.float32),
    mesh=_mesh,
    compiler_params=pltpu.CompilerParams(use_tc_tiling_on_sc=False),
    scratch_types=[
        pltpu.VMEM((_B_PER_W,), jnp.int32),
        pltpu.VMEM((_CHUNKS, _C, D_MODEL), jnp.float32),
        [pltpu.SemaphoreType.DMA] * _CHUNKS,
        [pltpu.SemaphoreType.DMA] * _CHUNKS,
    ],
)
def _sc_gather(table_hbm, idx_hbm, out_hbm, idx_v, rows_v, gsems, wsems):
    wid = lax.axis_index("s") * _NUM_CORES + lax.axis_index("c")
    base = wid * _B_PER_W
    pltpu.sync_copy(idx_hbm.at[pl.ds(base, _B_PER_W)], idx_v)
    gathers = [
        pltpu.async_copy(
            table_hbm.at[idx_v.at[pl.ds(c * _C, _C)]], rows_v.at[c], gsems[c]
        )
        for c in range(_CHUNKS)
    ]
    writes = []
    for c in range(_CHUNKS):
        gathers[c].wait()
        writes.append(
            pltpu.async_copy(
                rows_v.at[c],
                out_hbm.at[pl.ds(base + c * _C, _C), pl.ds(0, D_MODEL)],
                wsems[c],
            )
        )
    for w in writes:
        w.wait()


_MM_B = 2048


def _onehot_mm_body(pos_ref, table_ref, out_ref):
    pos = pos_ref[...].reshape(1, _MM_B)
    rows = lax.broadcasted_iota(jnp.int32, (MAX_LEN, _MM_B), 0)
    onehot = jnp.where(rows == pos, 1.0, 0.0).astype(jnp.float32)
    tab = table_ref[...]
    hi = tab.astype(jnp.bfloat16).astype(jnp.float32)
    lo = tab - hi
    dims = (((0,), (0,)), ((), ()))
    out_ref[...] = lax.dot_general(
        hi, onehot, dims, preferred_element_type=jnp.float32
    ) + lax.dot_general(
        lo, onehot, dims, preferred_element_type=jnp.float32
    )


_onehot_mm = pl.pallas_call(
    _onehot_mm_body,
    grid=(_B_TC // _MM_B,),
    in_specs=[
        pl.BlockSpec((_MM_B,), lambda i: (i + _B_SC // _MM_B,)),
        pl.BlockSpec((MAX_LEN, D_MODEL), lambda i: (0, 0)),
    ],
    out_specs=pl.BlockSpec((D_MODEL, _MM_B), lambda i: (0, i + _B_SC // _MM_B)),
    out_shape=jax.ShapeDtypeStruct((D_MODEL, BATCH), jnp.float32),
)

_TC_ROWS = 4096


def _xpose_body(out_t_ref, in_ref, out_ref):
    out_ref[...] = in_ref[:, :D_MODEL].T


_xpose_merge = pl.pallas_call(
    _xpose_body,
    grid=(_B_SC // _TC_ROWS,),
    in_specs=[
        pl.BlockSpec(memory_space=pl.ANY),
        pl.BlockSpec((_TC_ROWS, 128), lambda i: (i, 0)),
    ],
    out_specs=pl.BlockSpec((D_MODEL, _TC_ROWS), lambda i: (0, i)),
    out_shape=jax.ShapeDtypeStruct((D_MODEL, BATCH), jnp.float32),
    input_output_aliases={0: 0},
)


def kernel(positions, table):
    pos = positions.astype(jnp.int32)
    staged = _sc_gather(table, pos)
    out_t = _onehot_mm(pos, table)
    out_t = _xpose_merge(out_t, staged)
    return out_t.T


# speedup vs baseline: 1.0845x; 1.0845x over previous
import functools

import jax
import jax.numpy as jnp
from jax import lax
from jax.experimental import pallas as pl
from jax.experimental.pallas import tpu as pltpu
from jax.experimental.pallas import tpu_sc as plsc

MAX_LEN = 512
D_MODEL = 64
BATCH = 16384

_B_SC = 6144
_B_TC = BATCH - _B_SC

_NUM_CORES = 1
_NUM_SUBCORES = 16
_NUM_WORKERS = _NUM_CORES * _NUM_SUBCORES
_B_PER_W = _B_SC // _NUM_WORKERS

_CHUNKS = 2
_C = _B_PER_W // _CHUNKS

_mesh = plsc.VectorSubcoreMesh(
    core_axis_name="c", subcore_axis_name="s",
    num_cores=_NUM_CORES, num_subcores=_NUM_SUBCORES,
)


@functools.partial(
    pl.kernel,
    out_type=jax.ShapeDtypeStruct((_B_SC, 128), jnp.float32),
    mesh=_mesh,
    compiler_params=pltpu.CompilerParams(use_tc_tiling_on_sc=False),
    scratch_types=[
        pltpu.VMEM((_B_PER_W,), jnp.int32),
        pltpu.VMEM((_CHUNKS, _C, D_MODEL), jnp.float32),
        [pltpu.SemaphoreType.DMA] * _CHUNKS,
        [pltpu.SemaphoreType.DMA] * _CHUNKS,
    ],
)
def _sc_gather(table_hbm, idx_hbm, out_hbm, idx_v, rows_v, gsems, wsems):
    wid = lax.axis_index("s") * _NUM_CORES + lax.axis_index("c")
    base = wid * _B_PER_W
    pltpu.sync_copy(idx_hbm.at[pl.ds(base, _B_PER_W)], idx_v)
    gathers = [
        pltpu.async_copy(
            table_hbm.at[idx_v.at[pl.ds(c * _C, _C)]], rows_v.at[c], gsems[c]
        )
        for c in range(_CHUNKS)
    ]
    writes = []
    for c in range(_CHUNKS):
        gathers[c].wait()
        writes.append(
            pltpu.async_copy(
                rows_v.at[c],
                out_hbm.at[pl.ds(base + c * _C, _C), pl.ds(0, D_MODEL)],
                wsems[c],
            )
        )
    for w in writes:
        w.wait()


_MM_B = 2048


def _onehot_mm_body(pos_ref, table_ref, out_ref):
    pos = pos_ref[...].reshape(1, _MM_B)
    rows = lax.broadcasted_iota(jnp.int32, (MAX_LEN, _MM_B), 0)
    onehot = jnp.where(rows == pos, 1.0, 0.0).astype(jnp.float32)
    tab = table_ref[...]
    hi = tab.astype(jnp.bfloat16).astype(jnp.float32)
    lo = tab - hi
    dims = (((0,), (0,)), ((), ()))
    out_ref[...] = lax.dot_general(
        hi, onehot, dims, preferred_element_type=jnp.float32
    ) + lax.dot_general(
        lo, onehot, dims, preferred_element_type=jnp.float32
    )


_onehot_mm = pl.pallas_call(
    _onehot_mm_body,
    grid=(_B_TC // _MM_B,),
    in_specs=[
        pl.BlockSpec((_MM_B,), lambda i: (i + _B_SC // _MM_B,)),
        pl.BlockSpec((MAX_LEN, D_MODEL), lambda i: (0, 0)),
    ],
    out_specs=pl.BlockSpec((D_MODEL, _MM_B), lambda i: (0, i + _B_SC // _MM_B)),
    out_shape=jax.ShapeDtypeStruct((D_MODEL, BATCH), jnp.float32),
)

_TC_ROWS = 3072


def _xpose_body(out_t_ref, in_ref, out_ref):
    out_ref[...] = in_ref[:, :D_MODEL].T


_xpose_merge = pl.pallas_call(
    _xpose_body,
    grid=(_B_SC // _TC_ROWS,),
    in_specs=[
        pl.BlockSpec(memory_space=pl.ANY),
        pl.BlockSpec((_TC_ROWS, 128), lambda i: (i, 0)),
    ],
    out_specs=pl.BlockSpec((D_MODEL, _TC_ROWS), lambda i: (0, i)),
    out_shape=jax.ShapeDtypeStruct((D_MODEL, BATCH), jnp.float32),
    input_output_aliases={0: 0},
)


def kernel(positions, table):
    pos = positions.astype(jnp.int32)
    staged = _sc_gather(table, pos)
    out_t = _onehot_mm(pos, table)
    out_t = _xpose_merge(out_t, staged)
    return out_t.T
